# baseline (speedup 1.0000x reference)
import functools
import math

import jax
import jax.numpy as jnp
from jax import lax
from jax.experimental import pallas as pl
from jax.experimental.pallas import tpu as pltpu
from jax.experimental.pallas import tpu_sc as plsc

_info = plsc.get_sparse_core_info()
_NC, _NS, _L = _info.num_cores, _info.num_subcores, _info.num_lanes
_NW = _NC * _NS


def _make_lookup(BSZ: int, H: int, V: int, D: int, scale: float):
  assert BSZ % (_NW * 2) == 0 and D % _L == 0
  rows_per_w = BSZ // _NW
  n_outer = rows_per_w // 2
  mesh = plsc.VectorSubcoreMesh(core_axis_name="c", subcore_axis_name="s")

  @functools.partial(
      pl.kernel,
      mesh=mesh,
      out_type=jax.ShapeDtypeStruct((BSZ, H, D), jnp.float32),
      compiler_params=pltpu.CompilerParams(use_tc_tiling_on_sc=False),
      scratch_types=[
          [pltpu.VMEM((H,), jnp.int32)] * 2,
          [pltpu.VMEM((H, D), jnp.float32)] * 2,
          [pltpu.VMEM((H, D), jnp.float32)] * 2,
          [pltpu.SemaphoreType.DMA] * 2,
          [pltpu.SemaphoreType.DMA] * 2,
      ],
  )
  def lookup_kernel(table_hbm, seq_hbm, out_hbm, idx_v, rows_in, rows_out,
                    gsem, ssem):
    wid = lax.axis_index("s") * _NC + lax.axis_index("c")
    base = wid * rows_per_w

    for b in (0, 1):
      pltpu.sync_copy(seq_hbm.at[base + b], idx_v[b])
      pltpu.async_copy(table_hbm.at[idx_v[b]], rows_in[b], gsem[b])

    def outer_body(go, carry):
      for b in (0, 1):
        r = base + 2 * go + b
        pltpu.make_async_copy(table_hbm.at[idx_v[b]], rows_in[b],
                              gsem[b]).wait()
        @pl.when(go > 0)
        def _():
          pltpu.make_async_copy(rows_out[b], out_hbm.at[r], ssem[b]).wait()

        def scale_row(j, c2):
          for k in range(D // _L):
            sl = pl.ds(k * _L, _L)
            rows_out[b][j, sl] = rows_in[b][j, sl] * scale
          return c2

        lax.fori_loop(0, H, scale_row, 0, unroll=4)

        pltpu.async_copy(rows_out[b], out_hbm.at[r], ssem[b])

        @pl.when(go < n_outer - 1)
        def _():
          pltpu.sync_copy(seq_hbm.at[r + 2], idx_v[b])
          pltpu.async_copy(table_hbm.at[idx_v[b]], rows_in[b], gsem[b])

      return carry

    lax.fori_loop(0, n_outer, outer_body, 0)

    for b in (0, 1):
      pltpu.make_async_copy(rows_out[b], out_hbm.at[base], ssem[b]).wait()

  return lookup_kernel


def kernel(seq, table):
  bsz, hist = seq.shape
  V, D = table.shape
  return _make_lookup(bsz, hist, V, D, math.sqrt(D))(table, seq)
